# baseline (speedup 1.0000x reference)
import jax
import jax.numpy as jnp
from jax import lax
from jax.experimental import pallas as pl
from jax.experimental.pallas import tpu as pltpu
from jax.experimental.pallas import tpu_sc as plsc

F32 = jnp.float32

_NC = 2
_NS = 16
_CHUNK = 128
_NBUF = 2
_NPAD = 16384


def _sc_mesh():
    return plsc.VectorSubcoreMesh(core_axis_name="c", subcore_axis_name="s")


def _make_deg_kernel(nchunks, n_nodes):
    epw = _NPAD // _NS
    rows_w = epw // 128
    cps = nchunks // (_NC * _NS)

    def body(dst_hbm, out_hbm, ones_v, id2, t1d, t2d, sh, sem):
        c = lax.axis_index("c")
        s = lax.axis_index("s")
        w = c * _NS + s
        base = pl.multiple_of(s * epw, 8)

        pltpu.sync_copy(dst_hbm.at[pl.ds(pl.multiple_of(w * cps, 8), cps)], id2)

        @pl.loop(0, epw, step=16)
        def _(r):
            t1d[pl.ds(r, 16)] = jnp.zeros((16,), F32)

        @pl.loop(0, _CHUNK, step=16)
        def _(r):
            ones_v[pl.ds(r, 16)] = jnp.ones((16,), F32)

        pltpu.sync_copy(t1d, sh.at[pl.ds(base, epw)])
        plsc.subcore_barrier()

        @pl.loop(0, cps)
        def _(j):
            pltpu.async_copy(ones_v, sh.at[id2.at[j]], sem, add=True)

        @pl.loop(0, cps)
        def _(j):
            pltpu.make_async_copy(ones_v, sh.at[id2.at[0]], sem).wait()

        plsc.subcore_barrier()
        pltpu.sync_copy(sh.at[pl.ds(base, epw)], t1d)

        @pl.loop(0, rows_w)
        def _(r):
            @pl.loop(0, 128, step=16)
            def _(c0):
                t2d[r, pl.ds(c0, 16)] = t1d[pl.ds(r * 128 + c0, 16)]

        row0 = pl.multiple_of((c * _NPAD + s * epw) // 128, 8)
        pltpu.sync_copy(t2d, out_hbm.at[pl.ds(row0, rows_w)])

    return pl.kernel(
        body,
        out_type=jax.ShapeDtypeStruct((_NC * _NPAD // 128, 128), F32),
        mesh=_sc_mesh(),
        scratch_types=[
            pltpu.VMEM((_CHUNK,), F32),
            pltpu.VMEM((cps, _CHUNK), jnp.int32),
            pltpu.VMEM((epw,), F32),
            pltpu.VMEM((rows_w, 128), F32),
            pltpu.VMEM_SHARED((_NPAD,), F32),
            pltpu.SemaphoreType.DMA,
        ],
    )


def _make_edge_kernel(nchunks, n_nodes, d):
    step = (n_nodes // _NS) & ~7
    rw = n_nodes - (_NS - 1) * step
    n_acc = n_nodes + 8
    cps = nchunks // (_NC * _NS)
    half = cps // 2
    kmax = half // _NBUF

    def body(hs_hbm, src_hbm, dst_hbm, out_hbm,
             is2, id2, rv0, rv1, sh, g0, g1, s0, s1):
        c = lax.axis_index("c")
        s = lax.axis_index("s")
        w = c * _NS + s
        base = pl.multiple_of(s * step, 8)
        rvs = (rv0, rv1)
        gsem = (g0, g1)
        ssem = (s0, s1)
        zv = jnp.zeros((16,), F32)

        @pl.loop(0, _CHUNK)
        def _(r):
            @pl.loop(0, d, step=16)
            def _(c0):
                rv0[r, pl.ds(c0, 16)] = zv

        @pl.loop(0, rw // _CHUNK)
        def _(k):
            pltpu.sync_copy(
                rv0, sh.at[pl.ds(pl.multiple_of(base + k * _CHUNK, 8), _CHUNK)])

        plsc.subcore_barrier()

        for ph in range(2):
            cbase = pl.multiple_of(w * cps + ph * half, 8)
            pltpu.sync_copy(src_hbm.at[pl.ds(cbase, half)], is2)
            pltpu.sync_copy(dst_hbm.at[pl.ds(cbase, half)], id2)

            for b in range(_NBUF):
                pltpu.async_copy(hs_hbm.at[is2.at[b]], rvs[b], gsem[b])

            @pl.loop(0, kmax)
            def _(k):
                for b in range(_NBUF):
                    j = k * _NBUF + b
                    pltpu.make_async_copy(hs_hbm.at[is2.at[0]], rvs[b],
                                          gsem[b]).wait()
                    pltpu.async_copy(rvs[b], sh.at[id2.at[j]], ssem[b],
                                     add=True)

                    @pl.when(k < kmax - 1)
                    def _(b=b, j=j):
                        pltpu.make_async_copy(rvs[b], sh.at[id2.at[0]],
                                              ssem[b]).wait()
                        pltpu.async_copy(hs_hbm.at[is2.at[j + _NBUF]], rvs[b],
                                         gsem[b])

            for b in range(_NBUF):
                pltpu.make_async_copy(rvs[b], sh.at[id2.at[0]], ssem[b]).wait()

        plsc.subcore_barrier()
        pltpu.sync_copy(sh.at[pl.ds(base, rw)],
                        out_hbm.at[pl.ds(pl.multiple_of(c * n_nodes + base, 8), rw)])

    return pl.kernel(
        body,
        out_type=jax.ShapeDtypeStruct((_NC * n_nodes, d), F32),
        mesh=_sc_mesh(),
        scratch_types=[
            pltpu.VMEM((half, _CHUNK), jnp.int32),
            pltpu.VMEM((half, _CHUNK), jnp.int32),
            pltpu.VMEM((_CHUNK, d), F32),
            pltpu.VMEM((_CHUNK, d), F32),
            pltpu.VMEM_SHARED((n_acc, d), F32),
            pltpu.SemaphoreType.DMA,
            pltpu.SemaphoreType.DMA,
            pltpu.SemaphoreType.DMA,
            pltpu.SemaphoreType.DMA,
        ],
    )


def _mm1_body(x_ref, w1_ref, b1_ref, wc_ref, h_ref):
    t = lax.dot_general(x_ref[...], w1_ref[...], (((1,), (1,)), ((), ())),
                        preferred_element_type=F32)
    t = jnp.maximum(t + b1_ref[...], 0.0)
    h_ref[...] = lax.dot_general(t, wc_ref[...], (((1,), (1,)), ((), ())),
                                 preferred_element_type=F32)


def _dinv_body(dp_ref, o_ref):
    dp = dp_ref[...]
    npr = _NPAD // 128
    deg = dp[:npr] + dp[npr:] + 1.0
    o_ref[...] = lax.rsqrt(deg)


def _scale_body(h_ref, dv_ref, hs_ref):
    hs_ref[...] = h_ref[...] * dv_ref[...]


def _final_body(acca_ref, accb_ref, hs_ref, dv_ref, bc_ref, wo_ref, bo_ref,
                y_ref):
    acc = acca_ref[...] + accb_ref[...]
    a = jnp.maximum((acc + hs_ref[...]) * dv_ref[...] + bc_ref[...], 0.0)
    y_ref[...] = lax.dot_general(a, wo_ref[...], (((1,), (1,)), ((), ())),
                                 preferred_element_type=F32) + bo_ref[...]


def _row_spec(br, d):
    return pl.BlockSpec((br, d), lambda i: (i, 0))


def _full_spec(shape):
    nd = len(shape)
    return pl.BlockSpec(shape, lambda i: (0,) * nd)


def kernel(x, edge_index, W1, b1, Wc, bc, Wo, bo):
    n, d = x.shape
    e = edge_index.shape[1]
    br = 2000
    grid = (n // br,)

    nw = _NC * _NS
    nchunks0 = (e + _CHUNK - 1) // _CHUNK
    cps = ((nchunks0 + nw - 1) // nw + 2 * _NBUF - 1) // (2 * _NBUF) * 2 * _NBUF
    nchunks = nw * cps
    epad = nchunks * _CHUNK
    src_p = jnp.concatenate(
        [edge_index[0], jnp.zeros((epad - e,), jnp.int32)]).reshape(nchunks,
                                                                    _CHUNK)
    dst_p = jnp.concatenate(
        [edge_index[1], jnp.full((epad - e,), n, jnp.int32)]).reshape(nchunks,
                                                                      _CHUNK)
    b1r = b1.reshape(1, d)
    bcr = bc.reshape(1, d)
    bor = bo.reshape(1, d)

    deg_pad = _make_deg_kernel(nchunks, n)(dst_p)

    h = pl.pallas_call(
        _mm1_body,
        grid=grid,
        in_specs=[_row_spec(br, d), _full_spec((d, d)), _full_spec((1, d)),
                  _full_spec((d, d))],
        out_specs=_row_spec(br, d),
        out_shape=jax.ShapeDtypeStruct((n, d), F32),
    )(x, W1, b1r, Wc)

    npr = _NPAD // 128
    dinv_pad = pl.pallas_call(
        _dinv_body,
        grid=(1,),
        in_specs=[_full_spec((2 * npr, 128))],
        out_specs=_full_spec((npr, 128)),
        out_shape=jax.ShapeDtypeStruct((npr, 128), F32),
    )(deg_pad)
    dinv_col = dinv_pad.reshape(_NPAD)[:n].reshape(n, 1)

    hs = pl.pallas_call(
        _scale_body,
        grid=grid,
        in_specs=[_row_spec(br, d), _row_spec(br, 1)],
        out_specs=_row_spec(br, d),
        out_shape=jax.ShapeDtypeStruct((n, d), F32),
    )(h, dinv_col)

    acc2 = _make_edge_kernel(nchunks, n, d)(hs, src_p, dst_p)

    nb = n // br
    y = pl.pallas_call(
        _final_body,
        grid=grid,
        in_specs=[_row_spec(br, d),
                  pl.BlockSpec((br, d), lambda i, nb=nb: (i + nb, 0)),
                  _row_spec(br, d),
                  _row_spec(br, 1),
                  _full_spec((1, d)), _full_spec((d, d)), _full_spec((1, d))],
        out_specs=_row_spec(br, d),
        out_shape=jax.ShapeDtypeStruct((n, d), F32),
    )(acc2, acc2, hs, dinv_col, bcr, Wo, bor)

    return y


# speedup vs baseline: 3.2414x; 3.2414x over previous
import jax
import jax.numpy as jnp
from jax import lax
from jax.experimental import pallas as pl
from jax.experimental.pallas import tpu as pltpu
from jax.experimental.pallas import tpu_sc as plsc

F32 = jnp.float32

_NC = 2
_NS = 16
_CHUNK = 128
_NBUF = 2
_NPAD = 16384


def _sc_mesh():
    return plsc.VectorSubcoreMesh(core_axis_name="c", subcore_axis_name="s")


def _make_deg_kernel(nchunks, n_nodes):
    epw = _NPAD // _NS
    rows_w = epw // 128
    cps = nchunks // (_NC * _NS)

    def body(dst_hbm, out_hbm, ones_v, id2, t1d, t2d, sh, sem):
        c = lax.axis_index("c")
        s = lax.axis_index("s")
        w = c * _NS + s
        base = pl.multiple_of(s * epw, 8)

        pltpu.sync_copy(dst_hbm.at[pl.ds(pl.multiple_of(w * cps, 8), cps)], id2)

        @pl.loop(0, epw, step=16)
        def _(r):
            t1d[pl.ds(r, 16)] = jnp.zeros((16,), F32)

        @pl.loop(0, _CHUNK, step=16)
        def _(r):
            ones_v[pl.ds(r, 16)] = jnp.ones((16,), F32)

        pltpu.sync_copy(t1d, sh.at[pl.ds(base, epw)])
        plsc.subcore_barrier()

        @pl.loop(0, cps)
        def _(j):
            pltpu.async_copy(ones_v, sh.at[id2.at[j]], sem, add=True)

        @pl.loop(0, cps)
        def _(j):
            pltpu.make_async_copy(ones_v, sh.at[id2.at[0]], sem).wait()

        plsc.subcore_barrier()
        pltpu.sync_copy(sh.at[pl.ds(base, epw)], t1d)

        @pl.loop(0, rows_w)
        def _(r):
            @pl.loop(0, 128, step=16)
            def _(c0):
                t2d[r, pl.ds(c0, 16)] = t1d[pl.ds(r * 128 + c0, 16)]

        row0 = pl.multiple_of((c * _NPAD + s * epw) // 128, 8)
        pltpu.sync_copy(t2d, out_hbm.at[pl.ds(row0, rows_w)])

    return pl.kernel(
        body,
        out_type=jax.ShapeDtypeStruct((_NC * _NPAD // 128, 128), F32),
        mesh=_sc_mesh(),
        scratch_types=[
            pltpu.VMEM((_CHUNK,), F32),
            pltpu.VMEM((cps, _CHUNK), jnp.int32),
            pltpu.VMEM((epw,), F32),
            pltpu.VMEM((rows_w, 128), F32),
            pltpu.VMEM_SHARED((_NPAD,), F32),
            pltpu.SemaphoreType.DMA,
        ],
    )


def _make_edge_kernel(nchunks, n_nodes, d):
    step = (n_nodes // _NS) & ~7
    rw = n_nodes - (_NS - 1) * step
    n_acc = n_nodes
    cps = nchunks // (_NC * _NS)
    half = cps // 2
    kmax = half // _NBUF

    def body(hs_hbm, src_hbm, dst_hbm, out_hbm,
             is2, id2, rv0, rv1, sh, g0, g1, s0, s1):
        c = lax.axis_index("c")
        s = lax.axis_index("s")
        w = c * _NS + s
        base = pl.multiple_of(s * step, 8)
        rvs = (rv0, rv1)
        gsem = (g0, g1)
        ssem = (s0, s1)
        zv = jnp.zeros((16,), F32)

        @pl.loop(0, _CHUNK)
        def _(r):
            @pl.loop(0, d, step=16)
            def _(c0):
                rv0[r, pl.ds(c0, 16)] = zv

        @pl.loop(0, rw // _CHUNK)
        def _(k):
            pltpu.sync_copy(
                rv0, sh.at[pl.ds(pl.multiple_of(base + k * _CHUNK, 8), _CHUNK)])

        plsc.subcore_barrier()

        for ph in range(2):
            cbase = pl.multiple_of(w * cps + ph * half, 8)
            pltpu.sync_copy(src_hbm.at[pl.ds(cbase, half)], is2)
            pltpu.sync_copy(dst_hbm.at[pl.ds(cbase, half)], id2)

            for b in range(_NBUF):
                pltpu.async_copy(hs_hbm.at[is2.at[b]], rvs[b], gsem[b])

            @pl.loop(0, kmax)
            def _(k):
                for b in range(_NBUF):
                    j = k * _NBUF + b
                    pltpu.make_async_copy(hs_hbm.at[is2.at[0]], rvs[b],
                                          gsem[b]).wait()
                    pltpu.async_copy(rvs[b], sh.at[id2.at[j]], ssem[b],
                                     add=True)

                    @pl.when(k < kmax - 1)
                    def _(b=b, j=j):
                        pltpu.make_async_copy(rvs[b], sh.at[id2.at[0]],
                                              ssem[b]).wait()
                        pltpu.async_copy(hs_hbm.at[is2.at[j + _NBUF]], rvs[b],
                                         gsem[b])

            for b in range(_NBUF):
                pltpu.make_async_copy(rvs[b], sh.at[id2.at[0]], ssem[b]).wait()

        plsc.subcore_barrier()
        pltpu.sync_copy(sh.at[pl.ds(base, rw)],
                        out_hbm.at[pl.ds(pl.multiple_of(c * n_nodes + base, 8), rw)])

    return pl.kernel(
        body,
        out_type=jax.ShapeDtypeStruct((_NC * n_nodes, d), F32),
        mesh=_sc_mesh(),
        scratch_types=[
            pltpu.VMEM((half, _CHUNK), jnp.int32),
            pltpu.VMEM((half, _CHUNK), jnp.int32),
            pltpu.VMEM((_CHUNK, d), F32),
            pltpu.VMEM((_CHUNK, d), F32),
            pltpu.VMEM_SHARED((n_acc, d), F32),
            pltpu.SemaphoreType.DMA,
            pltpu.SemaphoreType.DMA,
            pltpu.SemaphoreType.DMA,
            pltpu.SemaphoreType.DMA,
        ],
    )


def _mm1_body(x_ref, w1_ref, b1_ref, wc_ref, h_ref):
    t = lax.dot_general(x_ref[...], w1_ref[...], (((1,), (1,)), ((), ())),
                        preferred_element_type=F32)
    t = jnp.maximum(t + b1_ref[...], 0.0)
    h_ref[...] = lax.dot_general(t, wc_ref[...], (((1,), (1,)), ((), ())),
                                 preferred_element_type=F32)


def _dinv_body(dp_ref, o_ref):
    dp = dp_ref[...]
    npr = _NPAD // 128
    deg = dp[:npr] + dp[npr:] + 1.0
    o_ref[...] = lax.rsqrt(deg)


def _scale_body(h_ref, dv_ref, hs_ref):
    hs_ref[...] = h_ref[...] * dv_ref[...]


def _final_body(acca_ref, accb_ref, hs_ref, dv_ref, bc_ref, wo_ref, bo_ref,
                y_ref):
    acc = acca_ref[...] + accb_ref[...]
    a = jnp.maximum((acc + hs_ref[...]) * dv_ref[...] + bc_ref[...], 0.0)
    y_ref[...] = lax.dot_general(a, wo_ref[...], (((1,), (1,)), ((), ())),
                                 preferred_element_type=F32) + bo_ref[...]


def _row_spec(br, d):
    return pl.BlockSpec((br, d), lambda i: (i, 0))


def _full_spec(shape):
    nd = len(shape)
    return pl.BlockSpec(shape, lambda i: (0,) * nd)


def kernel(x, edge_index, W1, b1, Wc, bc, Wo, bo):
    n, d = x.shape
    e = edge_index.shape[1]
    br = 2000
    grid = (n // br,)

    nw = _NC * _NS
    nchunks0 = (e + _CHUNK - 1) // _CHUNK
    cps = ((nchunks0 + nw - 1) // nw + 2 * _NBUF - 1) // (2 * _NBUF) * 2 * _NBUF
    nchunks = nw * cps
    epad = nchunks * _CHUNK
    ar = jnp.arange(epad - e, dtype=jnp.int32)
    src_p = jnp.concatenate(
        [edge_index[0], n + (ar % 8)]).reshape(nchunks, _CHUNK)
    dst_p = jnp.concatenate(
        [edge_index[1], ar % n]).reshape(nchunks, _CHUNK)
    dst_g = jnp.concatenate(
        [edge_index[1], n + (ar % (_NPAD - n))]).reshape(nchunks, _CHUNK)
    b1r = b1.reshape(1, d)
    bcr = bc.reshape(1, d)
    bor = bo.reshape(1, d)

    deg_pad = _make_deg_kernel(nchunks, n)(dst_g)

    h = pl.pallas_call(
        _mm1_body,
        grid=grid,
        in_specs=[_row_spec(br, d), _full_spec((d, d)), _full_spec((1, d)),
                  _full_spec((d, d))],
        out_specs=_row_spec(br, d),
        out_shape=jax.ShapeDtypeStruct((n, d), F32),
    )(x, W1, b1r, Wc)

    npr = _NPAD // 128
    dinv_pad = pl.pallas_call(
        _dinv_body,
        grid=(1,),
        in_specs=[_full_spec((2 * npr, 128))],
        out_specs=_full_spec((npr, 128)),
        out_shape=jax.ShapeDtypeStruct((npr, 128), F32),
    )(deg_pad)
    dinv_col = dinv_pad.reshape(_NPAD)[:n].reshape(n, 1)

    hs = pl.pallas_call(
        _scale_body,
        grid=grid,
        in_specs=[_row_spec(br, d), _row_spec(br, 1)],
        out_specs=_row_spec(br, d),
        out_shape=jax.ShapeDtypeStruct((n, d), F32),
    )(h, dinv_col)

    hs_pad = jnp.concatenate([hs, jnp.zeros((8, d), F32)])
    acc2 = _make_edge_kernel(nchunks, n, d)(hs_pad, src_p, dst_p)

    nb = n // br
    y = pl.pallas_call(
        _final_body,
        grid=grid,
        in_specs=[_row_spec(br, d),
                  pl.BlockSpec((br, d), lambda i, nb=nb: (i + nb, 0)),
                  _row_spec(br, d),
                  _row_spec(br, 1),
                  _full_spec((1, d)), _full_spec((d, d)), _full_spec((1, d))],
        out_specs=_row_spec(br, d),
        out_shape=jax.ShapeDtypeStruct((n, d), F32),
    )(acc2, acc2, hs, dinv_col, bcr, Wo, bor)

    return y
